# Initial kernel scaffold; baseline (speedup 1.0000x reference)
import jax
import jax.numpy as jnp
from jax.experimental import pallas as pl


def kernel(x, edge_index, W):
    raise NotImplementedError("write your pallas kernel here")


# speedup vs baseline: 4.3971x; 4.3971x over previous
import functools

import jax
import jax.numpy as jnp
from jax import lax
from jax.experimental import pallas as pl
from jax.experimental.pallas import tpu as pltpu
from jax.experimental.pallas import tpu_sc as plsc

N = 10000
E = 320000
D = 128
NPAD = 10240
CHUNK = 80
EDGES_PER_CORE = E // 2
EDGES_PER_SUB = EDGES_PER_CORE // 16
CHUNKS_PER_SUB = EDGES_PER_SUB // CHUNK
STRIPE = NPAD // 16

_MESH = plsc.VectorSubcoreMesh(core_axis_name="c", subcore_axis_name="s")


def _fill(ref, nrows, ncols, value):
    def body(r, carry):
        for j in range(ncols // 16):
            ref[r, pl.ds(j * 16, 16)] = jnp.full((16,), value, jnp.float32)
        return carry
    lax.fori_loop(0, nrows, body, 0)


@functools.partial(
    pl.kernel,
    mesh=_MESH,
    out_type=jax.ShapeDtypeStruct((2, NPAD, D), jnp.float32),
    scratch_types=[
        pltpu.VMEM((CHUNK,), jnp.int32),
        pltpu.VMEM((CHUNK, D), jnp.float32),
        pltpu.VMEM_SHARED((NPAD, D), jnp.float32),
    ],
)
def _sc_degree(ei_dst, cnt_out, dst_v, val_v, cnt_sh):
    c = lax.axis_index("c")
    s = lax.axis_index("s")
    _fill(val_v, CHUNK, D, 0.0)
    for j in range(STRIPE // CHUNK):
        pltpu.sync_copy(val_v, cnt_sh.at[pl.ds(s * STRIPE + j * CHUNK, CHUNK)])
    _fill(val_v, CHUNK, D, 1.0)
    plsc.subcore_barrier()

    def body(t, carry):
        base = c * EDGES_PER_CORE + s * EDGES_PER_SUB + t * CHUNK
        pltpu.sync_copy(ei_dst.at[pl.ds(base, CHUNK)], dst_v)
        pltpu.sync_copy(val_v, cnt_sh.at[dst_v], add=True)
        return carry

    lax.fori_loop(0, CHUNKS_PER_SUB, body, 0)
    plsc.subcore_barrier()
    pltpu.sync_copy(cnt_sh.at[pl.ds(s * STRIPE, STRIPE)],
                    cnt_out.at[c, pl.ds(s * STRIPE, STRIPE)])


@functools.partial(
    pl.kernel,
    mesh=_MESH,
    out_type=jax.ShapeDtypeStruct((2, NPAD, D), jnp.float32),
    scratch_types=[
        pltpu.VMEM((CHUNK,), jnp.int32),
        pltpu.VMEM((CHUNK,), jnp.int32),
        pltpu.VMEM((CHUNK, D), jnp.float32),
        pltpu.VMEM_SHARED((NPAD, D), jnp.float32),
        pltpu.SemaphoreType.DMA,
    ],
)
def _sc_round(h, ei_src, ei_dst, agg_out, src_v, dst_v, rows_v, agg_sh, sem):
    c = lax.axis_index("c")
    s = lax.axis_index("s")
    _fill(rows_v, CHUNK, D, 0.0)
    for j in range(STRIPE // CHUNK):
        pltpu.sync_copy(rows_v, agg_sh.at[pl.ds(s * STRIPE + j * CHUNK, CHUNK)])
    plsc.subcore_barrier()

    def body(t, carry):
        base = c * EDGES_PER_CORE + s * EDGES_PER_SUB + t * CHUNK
        pltpu.sync_copy(ei_src.at[pl.ds(base, CHUNK)], src_v)
        pltpu.sync_copy(ei_dst.at[pl.ds(base, CHUNK)], dst_v)
        pltpu.async_copy(h.at[src_v], rows_v, sem).wait()
        pltpu.sync_copy(rows_v, agg_sh.at[dst_v], add=True)
        return carry

    lax.fori_loop(0, CHUNKS_PER_SUB, body, 0)
    plsc.subcore_barrier()
    pltpu.sync_copy(agg_sh.at[pl.ds(s * STRIPE, STRIPE)],
                    agg_out.at[c, pl.ds(s * STRIPE, STRIPE)])


_R = 2000
_GRID = N // _R

_cnt_spec = pl.BlockSpec((2, _R, D), lambda i: (0, i, 0))
_p_spec = pl.BlockSpec((2, _R, D), lambda i: (0, i, 0))
_row_spec = pl.BlockSpec((_R, D), lambda i: (i, 0))


def _deg(cnt_blk):
    return jnp.maximum(cnt_blk[0, :, 0:1] + cnt_blk[1, :, 0:1], 1.0)


def _tc_in_body(x_ref, w_ref, cnt_ref, o_ref):
    xw = lax.dot_general(x_ref[...], w_ref[...], (((1,), (1,)), ((), ())),
                         preferred_element_type=jnp.float32)
    o_ref[...] = xw * lax.rsqrt(_deg(cnt_ref[...]))


def _tc_mid_body(p_ref, cnt_ref, o_ref):
    o_ref[...] = (p_ref[0] + p_ref[1]) / _deg(cnt_ref[...])


def _tc_fin_body(p_ref, cnt_ref, o_ref):
    o_ref[...] = (p_ref[0] + p_ref[1]) * lax.rsqrt(_deg(cnt_ref[...]))


_out_t = jax.ShapeDtypeStruct((N, D), jnp.float32)

_tc_in = pl.pallas_call(
    _tc_in_body, grid=(_GRID,),
    in_specs=[_row_spec, pl.BlockSpec((D, D), lambda i: (0, 0)), _cnt_spec],
    out_specs=_row_spec, out_shape=_out_t)

_tc_mid = pl.pallas_call(
    _tc_mid_body, grid=(_GRID,),
    in_specs=[_p_spec, _cnt_spec], out_specs=_row_spec, out_shape=_out_t)

_tc_fin = pl.pallas_call(
    _tc_fin_body, grid=(_GRID,),
    in_specs=[_p_spec, _cnt_spec], out_specs=_row_spec, out_shape=_out_t)


def kernel(x, edge_index, W):
    src = edge_index[0]
    dst = edge_index[1]
    cnt = _sc_degree(dst)
    x1 = _tc_in(x, W, cnt)
    p = _sc_round(x1, src, dst)
    x2 = _tc_mid(p, cnt)
    q = _sc_round(x2, src, dst)
    return _tc_fin(q, cnt)


# speedup vs baseline: 7.6841x; 1.7475x over previous
import functools

import jax
import jax.numpy as jnp
from jax import lax
from jax.experimental import pallas as pl
from jax.experimental.pallas import tpu as pltpu
from jax.experimental.pallas import tpu_sc as plsc

N = 10000
E = 320000
D = 128
NPAD = 10240
CHUNK = 80
EDGES_PER_CORE = E // 2
EDGES_PER_SUB = EDGES_PER_CORE // 16
CHUNKS_PER_SUB = EDGES_PER_SUB // CHUNK
SECS = 5
CPS = CHUNKS_PER_SUB // SECS
EPS = CPS * CHUNK
STRIPE = NPAD // 16

_MESH = plsc.VectorSubcoreMesh(core_axis_name="c", subcore_axis_name="s")


def _fill(ref, nrows, ncols, value):
    def body(r, carry):
        for j in range(ncols // 16):
            ref[r, pl.ds(j * 16, 16)] = jnp.full((16,), value, jnp.float32)
        return carry
    lax.fori_loop(0, nrows, body, 0)


def _load_idx2(idx_all, idx2):
    def body(t, carry):
        for j in range(CHUNK // 16):
            idx2[t, pl.ds(j * 16, 16)] = idx_all[pl.ds(t * CHUNK + j * 16, 16)]
        return carry
    lax.fori_loop(0, CPS, body, 0)


@functools.partial(
    pl.kernel,
    mesh=_MESH,
    out_type=jax.ShapeDtypeStruct((2, NPAD, D), jnp.float32),
    scratch_types=[
        pltpu.VMEM((EPS,), jnp.int32),
        pltpu.VMEM((CPS, CHUNK), jnp.int32),
        pltpu.VMEM((CHUNK, D), jnp.float32),
        pltpu.VMEM_SHARED((NPAD, D), jnp.float32),
        pltpu.SemaphoreType.DMA,
    ],
)
def _sc_degree(ei_dst, cnt_out, dst_sec, dst2, val_v, cnt_sh, sem):
    c = lax.axis_index("c")
    s = lax.axis_index("s")
    base = (c * 16 + s) * EDGES_PER_SUB
    _fill(val_v, CHUNK, D, 0.0)
    for j in range(STRIPE // CHUNK):
        pltpu.sync_copy(val_v, cnt_sh.at[pl.ds(s * STRIPE + j * CHUNK, CHUNK)])
    _fill(val_v, CHUNK, D, 1.0)
    plsc.subcore_barrier()

    def section(sec, carry):
        pltpu.sync_copy(ei_dst.at[pl.ds(base + sec * EPS, EPS)], dst_sec)
        _load_idx2(dst_sec, dst2)
        def fire(t, carry2):
            pltpu.async_copy(val_v, cnt_sh.at[dst2.at[t]], sem, add=True)
            return carry2
        lax.fori_loop(0, CPS, fire, 0)
        def drain(t, carry2):
            pltpu.make_async_copy(val_v, cnt_sh.at[dst2.at[t]], sem).wait()
            return carry2
        lax.fori_loop(0, CPS, drain, 0)
        return carry
    lax.fori_loop(0, SECS, section, 0)
    plsc.subcore_barrier()
    pltpu.sync_copy(cnt_sh.at[pl.ds(s * STRIPE, STRIPE)],
                    cnt_out.at[c, pl.ds(s * STRIPE, STRIPE)])


@functools.partial(
    pl.kernel,
    mesh=_MESH,
    out_type=jax.ShapeDtypeStruct((2, NPAD, D), jnp.float32),
    scratch_types=[
        pltpu.VMEM((EPS,), jnp.int32),
        pltpu.VMEM((EPS,), jnp.int32),
        pltpu.VMEM((CPS, CHUNK), jnp.int32),
        pltpu.VMEM((CHUNK, D), jnp.float32),
        pltpu.VMEM((CHUNK, D), jnp.float32),
        pltpu.VMEM_SHARED((NPAD, D), jnp.float32),
        pltpu.SemaphoreType.DMA,
        pltpu.SemaphoreType.DMA,
    ],
)
def _sc_round(h, ei_src, ei_dst, agg_out,
              src_sec, dst_sec, dst2, rows_a, rows_b, agg_sh, sem_a, sem_b):
    c = lax.axis_index("c")
    s = lax.axis_index("s")
    base = (c * 16 + s) * EDGES_PER_SUB
    _fill(rows_a, CHUNK, D, 0.0)
    for j in range(STRIPE // CHUNK):
        pltpu.sync_copy(rows_a, agg_sh.at[pl.ds(s * STRIPE + j * CHUNK, CHUNK)])
    plsc.subcore_barrier()

    def gather(t, rows, sem):
        return pltpu.async_copy(h.at[src_sec.at[pl.ds(t * CHUNK, CHUNK)]],
                                rows, sem)

    def gwait(t, rows, sem):
        pltpu.make_async_copy(h.at[src_sec.at[pl.ds(t * CHUNK, CHUNK)]],
                              rows, sem).wait()

    def scatter(t, rows):
        pltpu.sync_copy(rows, agg_sh.at[dst2.at[t]], add=True)

    def section(sec, carry):
        pltpu.sync_copy(ei_src.at[pl.ds(base + sec * EPS, EPS)], src_sec)
        pltpu.sync_copy(ei_dst.at[pl.ds(base + sec * EPS, EPS)], dst_sec)
        _load_idx2(dst_sec, dst2)
        gather(0, rows_a, sem_a)

        def body(g, carry2):
            t0 = 2 * g
            gwait(t0, rows_a, sem_a)
            gather(t0 + 1, rows_b, sem_b)
            scatter(t0, rows_a)
            gwait(t0 + 1, rows_b, sem_b)
            gather(t0 + 2, rows_a, sem_a)
            scatter(t0 + 1, rows_b)
            return carry2

        lax.fori_loop(0, (CPS - 1) // 2, body, 0)
        gwait(CPS - 1, rows_a, sem_a)
        scatter(CPS - 1, rows_a)
        return carry

    lax.fori_loop(0, SECS, section, 0)
    plsc.subcore_barrier()
    pltpu.sync_copy(agg_sh.at[pl.ds(s * STRIPE, STRIPE)],
                    agg_out.at[c, pl.ds(s * STRIPE, STRIPE)])


_R = 2000
_GRID = N // _R

_cnt_spec = pl.BlockSpec((2, _R, D), lambda i: (0, i, 0))
_p_spec = pl.BlockSpec((2, _R, D), lambda i: (0, i, 0))
_row_spec = pl.BlockSpec((_R, D), lambda i: (i, 0))


def _deg(cnt_blk):
    return jnp.maximum(cnt_blk[0, :, 0:1] + cnt_blk[1, :, 0:1], 1.0)


def _tc_in_body(x_ref, w_ref, cnt_ref, o_ref):
    xw = lax.dot_general(x_ref[...], w_ref[...], (((1,), (1,)), ((), ())),
                         preferred_element_type=jnp.float32)
    o_ref[...] = xw * lax.rsqrt(_deg(cnt_ref[...]))


def _tc_mid_body(p_ref, cnt_ref, o_ref):
    o_ref[...] = (p_ref[0] + p_ref[1]) / _deg(cnt_ref[...])


def _tc_fin_body(p_ref, cnt_ref, o_ref):
    o_ref[...] = (p_ref[0] + p_ref[1]) * lax.rsqrt(_deg(cnt_ref[...]))


_out_t = jax.ShapeDtypeStruct((N, D), jnp.float32)

_tc_in = pl.pallas_call(
    _tc_in_body, grid=(_GRID,),
    in_specs=[_row_spec, pl.BlockSpec((D, D), lambda i: (0, 0)), _cnt_spec],
    out_specs=_row_spec, out_shape=_out_t)

_tc_mid = pl.pallas_call(
    _tc_mid_body, grid=(_GRID,),
    in_specs=[_p_spec, _cnt_spec], out_specs=_row_spec, out_shape=_out_t)

_tc_fin = pl.pallas_call(
    _tc_fin_body, grid=(_GRID,),
    in_specs=[_p_spec, _cnt_spec], out_specs=_row_spec, out_shape=_out_t)


def kernel(x, edge_index, W):
    src = edge_index[0]
    dst = edge_index[1]
    cnt = _sc_degree(dst)
    x1 = _tc_in(x, W, cnt)
    p = _sc_round(x1, src, dst)
    x2 = _tc_mid(p, cnt)
    q = _sc_round(x2, src, dst)
    return _tc_fin(q, cnt)


# speedup vs baseline: 7.7181x; 1.0044x over previous
import functools

import jax
import jax.numpy as jnp
from jax import lax
from jax.experimental import pallas as pl
from jax.experimental.pallas import tpu as pltpu
from jax.experimental.pallas import tpu_sc as plsc

N = 10000
E = 320000
D = 128
NPAD = 10240
CHUNK = 80
EDGES_PER_CORE = E // 2
EDGES_PER_SUB = EDGES_PER_CORE // 16
CHUNKS_PER_SUB = EDGES_PER_SUB // CHUNK
SECS = 5
CPS = CHUNKS_PER_SUB // SECS
EPS = CPS * CHUNK
STRIPE = NPAD // 16

_MESH = plsc.VectorSubcoreMesh(core_axis_name="c", subcore_axis_name="s")


def _fill(ref, nrows, ncols, value):
    def body(r, carry):
        for j in range(ncols // 16):
            ref[r, pl.ds(j * 16, 16)] = jnp.full((16,), value, jnp.float32)
        return carry
    lax.fori_loop(0, nrows, body, 0)


def _load_idx2(idx_all, idx2):
    def body(t, carry):
        for j in range(CHUNK // 16):
            idx2[t, pl.ds(j * 16, 16)] = idx_all[pl.ds(t * CHUNK + j * 16, 16)]
        return carry
    lax.fori_loop(0, CPS, body, 0)


@functools.partial(
    pl.kernel,
    mesh=_MESH,
    out_type=jax.ShapeDtypeStruct((2, NPAD, D), jnp.float32),
    scratch_types=[
        pltpu.VMEM((EPS,), jnp.int32),
        pltpu.VMEM((CPS, CHUNK), jnp.int32),
        pltpu.VMEM((CHUNK, D), jnp.float32),
        pltpu.VMEM_SHARED((NPAD, D), jnp.float32),
        pltpu.SemaphoreType.DMA,
    ],
)
def _sc_degree(ei_dst, cnt_out, dst_sec, dst2, val_v, cnt_sh, sem):
    c = lax.axis_index("c")
    s = lax.axis_index("s")
    base = (c * 16 + s) * EDGES_PER_SUB
    _fill(val_v, CHUNK, D, 0.0)
    for j in range(STRIPE // CHUNK):
        pltpu.sync_copy(val_v, cnt_sh.at[pl.ds(s * STRIPE + j * CHUNK, CHUNK)])
    _fill(val_v, CHUNK, D, 1.0)
    plsc.subcore_barrier()

    def section(sec, carry):
        pltpu.sync_copy(ei_dst.at[pl.ds(base + sec * EPS, EPS)], dst_sec)
        _load_idx2(dst_sec, dst2)
        def fire(t, carry2):
            pltpu.async_copy(val_v, cnt_sh.at[dst2.at[t]], sem, add=True)
            return carry2
        lax.fori_loop(0, CPS, fire, 0)
        def drain(t, carry2):
            pltpu.make_async_copy(val_v, cnt_sh.at[dst2.at[t]], sem).wait()
            return carry2
        lax.fori_loop(0, CPS, drain, 0)
        return carry
    lax.fori_loop(0, SECS, section, 0)
    plsc.subcore_barrier()
    pltpu.sync_copy(cnt_sh.at[pl.ds(s * STRIPE, STRIPE)],
                    cnt_out.at[c, pl.ds(s * STRIPE, STRIPE)])


@functools.partial(
    pl.kernel,
    mesh=_MESH,
    out_type=jax.ShapeDtypeStruct((2, NPAD, D), jnp.float32),
    scratch_types=[
        pltpu.VMEM((EPS,), jnp.int32),
        pltpu.VMEM((EPS,), jnp.int32),
        pltpu.VMEM((CPS, CHUNK), jnp.int32),
        pltpu.VMEM((CHUNK, D), jnp.float32),
        pltpu.VMEM((CHUNK, D), jnp.float32),
        pltpu.VMEM_SHARED((NPAD, D), jnp.float32),
        pltpu.SemaphoreType.DMA,
        pltpu.SemaphoreType.DMA,
        pltpu.SemaphoreType.DMA,
        pltpu.SemaphoreType.DMA,
    ],
)
def _sc_round(h, ei_src, ei_dst, agg_out,
              src_sec, dst_sec, dst2, rows_a, rows_b, agg_sh,
              sem_a, sem_b, ssem_a, ssem_b):
    c = lax.axis_index("c")
    s = lax.axis_index("s")
    base = (c * 16 + s) * EDGES_PER_SUB
    _fill(rows_a, CHUNK, D, 0.0)
    for j in range(STRIPE // CHUNK):
        pltpu.sync_copy(rows_a, agg_sh.at[pl.ds(s * STRIPE + j * CHUNK, CHUNK)])
    plsc.subcore_barrier()

    def gather(t, rows, sem):
        return pltpu.async_copy(h.at[src_sec.at[pl.ds(t * CHUNK, CHUNK)]],
                                rows, sem)

    def gwait(t, rows, sem):
        pltpu.make_async_copy(h.at[src_sec.at[pl.ds(t * CHUNK, CHUNK)]],
                              rows, sem).wait()

    def sissue(t, rows, sem):
        pltpu.async_copy(rows, agg_sh.at[dst2.at[t]], sem, add=True)

    def swait(t, rows, sem):
        pltpu.make_async_copy(rows, agg_sh.at[dst2.at[t]], sem).wait()

    def section(sec, carry):
        pltpu.sync_copy(ei_src.at[pl.ds(base + sec * EPS, EPS)], src_sec)
        pltpu.sync_copy(ei_dst.at[pl.ds(base + sec * EPS, EPS)], dst_sec)
        _load_idx2(dst_sec, dst2)
        gather(0, rows_a, sem_a)
        gather(1, rows_b, sem_b)

        def body(g, carry2):
            t0 = 2 * g
            gwait(t0, rows_a, sem_a)
            sissue(t0, rows_a, ssem_a)
            gwait(t0 + 1, rows_b, sem_b)
            sissue(t0 + 1, rows_b, ssem_b)
            swait(t0, rows_a, ssem_a)
            gather(t0 + 2, rows_a, sem_a)
            swait(t0 + 1, rows_b, ssem_b)

            @pl.when(t0 + 3 < CPS)
            def _():
                gather(t0 + 3, rows_b, sem_b)
            return carry2

        lax.fori_loop(0, (CPS - 1) // 2, body, 0)
        gwait(CPS - 1, rows_a, sem_a)
        sissue(CPS - 1, rows_a, ssem_a)
        swait(CPS - 1, rows_a, ssem_a)
        return carry

    lax.fori_loop(0, SECS, section, 0)
    plsc.subcore_barrier()
    pltpu.sync_copy(agg_sh.at[pl.ds(s * STRIPE, STRIPE)],
                    agg_out.at[c, pl.ds(s * STRIPE, STRIPE)])


_R = 2000
_GRID = N // _R

_cnt_spec = pl.BlockSpec((2, _R, D), lambda i: (0, i, 0))
_p_spec = pl.BlockSpec((2, _R, D), lambda i: (0, i, 0))
_row_spec = pl.BlockSpec((_R, D), lambda i: (i, 0))


def _deg(cnt_blk):
    return jnp.maximum(cnt_blk[0, :, 0:1] + cnt_blk[1, :, 0:1], 1.0)


def _tc_in_body(x_ref, w_ref, cnt_ref, o_ref):
    xw = lax.dot_general(x_ref[...], w_ref[...], (((1,), (1,)), ((), ())),
                         preferred_element_type=jnp.float32)
    o_ref[...] = xw * lax.rsqrt(_deg(cnt_ref[...]))


def _tc_mid_body(p_ref, cnt_ref, o_ref):
    o_ref[...] = (p_ref[0] + p_ref[1]) / _deg(cnt_ref[...])


def _tc_fin_body(p_ref, cnt_ref, o_ref):
    o_ref[...] = (p_ref[0] + p_ref[1]) * lax.rsqrt(_deg(cnt_ref[...]))


_out_t = jax.ShapeDtypeStruct((N, D), jnp.float32)

_tc_in = pl.pallas_call(
    _tc_in_body, grid=(_GRID,),
    in_specs=[_row_spec, pl.BlockSpec((D, D), lambda i: (0, 0)), _cnt_spec],
    out_specs=_row_spec, out_shape=_out_t)

_tc_mid = pl.pallas_call(
    _tc_mid_body, grid=(_GRID,),
    in_specs=[_p_spec, _cnt_spec], out_specs=_row_spec, out_shape=_out_t)

_tc_fin = pl.pallas_call(
    _tc_fin_body, grid=(_GRID,),
    in_specs=[_p_spec, _cnt_spec], out_specs=_row_spec, out_shape=_out_t)


def kernel(x, edge_index, W):
    src = edge_index[0]
    dst = edge_index[1]
    cnt = _sc_degree(dst)
    x1 = _tc_in(x, W, cnt)
    p = _sc_round(x1, src, dst)
    x2 = _tc_mid(p, cnt)
    q = _sc_round(x2, src, dst)
    return _tc_fin(q, cnt)


# speedup vs baseline: 9.3691x; 1.2139x over previous
import functools

import jax
import jax.numpy as jnp
from jax import lax
from jax.experimental import pallas as pl
from jax.experimental.pallas import tpu as pltpu
from jax.experimental.pallas import tpu_sc as plsc

N = 10000
E = 320000
D = 128
NPAD = 10240
CHUNK = 80
EDGES_PER_CORE = E // 2
EDGES_PER_SUB = EDGES_PER_CORE // 16
CHUNKS_PER_SUB = EDGES_PER_SUB // CHUNK
SECS = 5
CPS = CHUNKS_PER_SUB // SECS
EPS = CPS * CHUNK
STRIPE = NPAD // 16

_MESH = plsc.VectorSubcoreMesh(core_axis_name="c", subcore_axis_name="s")


def _fill(ref, nrows, ncols, value):
    def body(r, carry):
        for j in range(ncols // 16):
            ref[r, pl.ds(j * 16, 16)] = jnp.full((16,), value, jnp.float32)
        return carry
    lax.fori_loop(0, nrows, body, 0)


def _load_idx2(idx_all, idx2):
    def body(t, carry):
        for j in range(CHUNK // 16):
            idx2[t, pl.ds(j * 16, 16)] = idx_all[pl.ds(t * CHUNK + j * 16, 16)]
        return carry
    lax.fori_loop(0, CPS, body, 0)


@functools.partial(
    pl.kernel,
    mesh=_MESH,
    out_type=jax.ShapeDtypeStruct((2, NPAD, D), jnp.float32),
    scratch_types=[
        pltpu.VMEM((EPS,), jnp.int32),
        pltpu.VMEM((CPS, CHUNK), jnp.int32),
        pltpu.VMEM((CHUNK, D), jnp.float32),
        pltpu.VMEM_SHARED((NPAD, D), jnp.float32),
        pltpu.SemaphoreType.DMA,
    ],
)
def _sc_degree(ei_dst, cnt_out, dst_sec, dst2, val_v, cnt_sh, sem):
    c = lax.axis_index("c")
    s = lax.axis_index("s")
    base = (c * 16 + s) * EDGES_PER_SUB
    _fill(val_v, CHUNK, D, 0.0)
    for j in range(STRIPE // CHUNK):
        pltpu.sync_copy(val_v, cnt_sh.at[pl.ds(s * STRIPE + j * CHUNK, CHUNK)])
    _fill(val_v, CHUNK, D, 1.0)
    plsc.subcore_barrier()

    def section(sec, carry):
        pltpu.sync_copy(ei_dst.at[pl.ds(base + sec * EPS, EPS)], dst_sec)
        _load_idx2(dst_sec, dst2)
        def fire(t, carry2):
            pltpu.async_copy(val_v, cnt_sh.at[dst2.at[t]], sem, add=True)
            return carry2
        lax.fori_loop(0, CPS, fire, 0)
        def drain(t, carry2):
            pltpu.make_async_copy(val_v, cnt_sh.at[dst2.at[t]], sem).wait()
            return carry2
        lax.fori_loop(0, CPS, drain, 0)
        return carry
    lax.fori_loop(0, SECS, section, 0)
    plsc.subcore_barrier()
    pltpu.sync_copy(cnt_sh.at[pl.ds(s * STRIPE, STRIPE)],
                    cnt_out.at[c, pl.ds(s * STRIPE, STRIPE)])


@functools.partial(
    pl.kernel,
    mesh=_MESH,
    out_type=jax.ShapeDtypeStruct((2, NPAD, D), jnp.float32),
    scratch_types=[
        pltpu.VMEM((EPS,), jnp.int32),
        pltpu.VMEM((EPS,), jnp.int32),
        pltpu.VMEM((CPS, CHUNK), jnp.int32),
        pltpu.VMEM((CHUNK, D), jnp.float32),
        pltpu.VMEM((CHUNK, D), jnp.float32),
        pltpu.VMEM((CHUNK, D), jnp.float32),
        pltpu.VMEM((CHUNK, D), jnp.float32),
        pltpu.VMEM_SHARED((NPAD, D), jnp.float32),
        pltpu.SemaphoreType.DMA,
        pltpu.SemaphoreType.DMA,
        pltpu.SemaphoreType.DMA,
        pltpu.SemaphoreType.DMA,
        pltpu.SemaphoreType.DMA,
        pltpu.SemaphoreType.DMA,
        pltpu.SemaphoreType.DMA,
        pltpu.SemaphoreType.DMA,
    ],
)
def _sc_round(h, ei_src, ei_dst, agg_out,
              src_sec, dst_sec, dst2, rows_a, rows_b, rows_c, rows_d, agg_sh,
              sem_a, sem_b, sem_c, sem_d, ssem_a, ssem_b, ssem_c, ssem_d):
    c = lax.axis_index("c")
    s = lax.axis_index("s")
    base = (c * 16 + s) * EDGES_PER_SUB
    _fill(rows_a, CHUNK, D, 0.0)
    for j in range(STRIPE // CHUNK):
        pltpu.sync_copy(rows_a, agg_sh.at[pl.ds(s * STRIPE + j * CHUNK, CHUNK)])
    plsc.subcore_barrier()

    def gather(t, rows, sem):
        return pltpu.async_copy(h.at[src_sec.at[pl.ds(t * CHUNK, CHUNK)]],
                                rows, sem)

    def gwait(t, rows, sem):
        pltpu.make_async_copy(h.at[src_sec.at[pl.ds(t * CHUNK, CHUNK)]],
                              rows, sem).wait()

    def sissue(t, rows, sem):
        pltpu.async_copy(rows, agg_sh.at[dst2.at[t]], sem, add=True)

    def swait(t, rows, sem):
        pltpu.make_async_copy(rows, agg_sh.at[dst2.at[t]], sem).wait()

    bufs = (rows_a, rows_b, rows_c, rows_d)
    gsems = (sem_a, sem_b, sem_c, sem_d)
    ssems = (ssem_a, ssem_b, ssem_c, ssem_d)

    def section(sec, carry):
        pltpu.sync_copy(ei_src.at[pl.ds(base + sec * EPS, EPS)], src_sec)
        pltpu.sync_copy(ei_dst.at[pl.ds(base + sec * EPS, EPS)], dst_sec)
        _load_idx2(dst_sec, dst2)
        for k in range(4):
            gather(k, bufs[k], gsems[k])

        def body(g, carry2):
            t0 = 4 * g
            for k in range(4):
                gwait(t0 + k, bufs[k], gsems[k])
                sissue(t0 + k, bufs[k], ssems[k])
            for k in range(4):
                swait(t0 + k, bufs[k], ssems[k])

                @pl.when(t0 + k + 4 < CPS)
                def _(tk=t0 + k + 4, bk=bufs[k], gk=gsems[k]):
                    gather(tk, bk, gk)
            return carry2

        lax.fori_loop(0, CPS // 4, body, 0)
        t_last = CPS - 1
        gwait(t_last, bufs[t_last % 4], gsems[t_last % 4])
        sissue(t_last, bufs[t_last % 4], ssems[t_last % 4])
        swait(t_last, bufs[t_last % 4], ssems[t_last % 4])
        return carry

    lax.fori_loop(0, SECS, section, 0)
    plsc.subcore_barrier()
    pltpu.sync_copy(agg_sh.at[pl.ds(s * STRIPE, STRIPE)],
                    agg_out.at[c, pl.ds(s * STRIPE, STRIPE)])


_R = 2000
_GRID = N // _R

_cnt_spec = pl.BlockSpec((2, _R, D), lambda i: (0, i, 0))
_p_spec = pl.BlockSpec((2, _R, D), lambda i: (0, i, 0))
_row_spec = pl.BlockSpec((_R, D), lambda i: (i, 0))


def _deg(cnt_blk):
    return jnp.maximum(cnt_blk[0, :, 0:1] + cnt_blk[1, :, 0:1], 1.0)


def _tc_in_body(x_ref, w_ref, cnt_ref, o_ref):
    xw = lax.dot_general(x_ref[...], w_ref[...], (((1,), (1,)), ((), ())),
                         preferred_element_type=jnp.float32)
    o_ref[...] = xw * lax.rsqrt(_deg(cnt_ref[...]))


def _tc_mid_body(p_ref, cnt_ref, o_ref):
    o_ref[...] = (p_ref[0] + p_ref[1]) / _deg(cnt_ref[...])


def _tc_fin_body(p_ref, cnt_ref, o_ref):
    o_ref[...] = (p_ref[0] + p_ref[1]) * lax.rsqrt(_deg(cnt_ref[...]))


_out_t = jax.ShapeDtypeStruct((N, D), jnp.float32)

_tc_in = pl.pallas_call(
    _tc_in_body, grid=(_GRID,),
    in_specs=[_row_spec, pl.BlockSpec((D, D), lambda i: (0, 0)), _cnt_spec],
    out_specs=_row_spec, out_shape=_out_t)

_tc_mid = pl.pallas_call(
    _tc_mid_body, grid=(_GRID,),
    in_specs=[_p_spec, _cnt_spec], out_specs=_row_spec, out_shape=_out_t)

_tc_fin = pl.pallas_call(
    _tc_fin_body, grid=(_GRID,),
    in_specs=[_p_spec, _cnt_spec], out_specs=_row_spec, out_shape=_out_t)


def kernel(x, edge_index, W):
    src = edge_index[0]
    dst = edge_index[1]
    cnt = _sc_degree(dst)
    x1 = _tc_in(x, W, cnt)
    p = _sc_round(x1, src, dst)
    x2 = _tc_mid(p, cnt)
    q = _sc_round(x2, src, dst)
    return _tc_fin(q, cnt)
